# initial kernel scaffold (unmeasured)
import jax
import jax.numpy as jnp
from jax import lax
from jax.experimental import pallas as pl
from jax.experimental.pallas import tpu as pltpu

N_DEV = 4
M_PER = 8192
MC = M_PER // N_DEV
K = 2048
N_OUT = 2048


def kernel(t, W):
    tb = t.astype(jnp.bfloat16)
    wb = W.astype(jnp.bfloat16)

    def body(t_hbm, w_ref, out_ref, stage, send_bufs, recv_bufs,
             rs_send_sems, rs_recv_sems, ag_send_sems, ag_recv_sems,
             local_sem):
        my = lax.axis_index("i")
        right = jnp.mod(my + 1, N_DEV)
        left = jnp.mod(my + N_DEV - 1, N_DEV)

        barrier_sem = pltpu.get_barrier_semaphore()
        for nbr in (left, right):
            pl.semaphore_signal(
                barrier_sem, inc=1,
                device_id=(nbr,), device_id_type=pl.DeviceIdType.MESH,
            )
        pl.semaphore_wait(barrier_sem, 2)

        def start_stage(c):
            cp = pltpu.make_async_copy(
                t_hbm.at[pl.ds(c * MC, MC)], stage, local_sem)
            cp.start()
            return cp

        cp0 = pltpu.make_async_copy(
            t_hbm.at[pl.ds(my * MC, MC)], send_bufs.at[0], local_sem)
        cp0.start()
        cp0.wait()

        for s in range(N_DEV - 1):
            rdma = pltpu.make_async_remote_copy(
                src_ref=send_bufs.at[s],
                dst_ref=recv_bufs.at[s],
                send_sem=rs_send_sems.at[s],
                recv_sem=rs_recv_sems.at[s],
                device_id=(right,),
                device_id_type=pl.DeviceIdType.MESH,
            )
            rdma.start()
            cp = start_stage(jnp.mod(my + N_DEV - s - 1, N_DEV))
            cp.wait()
            rdma.wait()
            partial = recv_bufs[s] + stage[...]
            if s < N_DEV - 2:
                send_bufs[s + 1] = partial
            else:
                acc = jnp.dot(partial, w_ref[...],
                              preferred_element_type=jnp.float32)
                own_c = jnp.mod(my + 1, N_DEV)
                out_ref[pl.ds(own_c * MC, MC)] = acc.astype(jnp.bfloat16)

        for h in range(N_DEV - 1):
            c_send = jnp.mod(my + 1 - h + N_DEV, N_DEV)
            rdma = pltpu.make_async_remote_copy(
                src_ref=out_ref.at[pl.ds(c_send * MC, MC)],
                dst_ref=out_ref.at[pl.ds(c_send * MC, MC)],
                send_sem=ag_send_sems.at[h],
                recv_sem=ag_recv_sems.at[h],
                device_id=(right,),
                device_id_type=pl.DeviceIdType.MESH,
            )
            rdma.start()
            rdma.wait()

    return pl.pallas_call(
        body,
        out_shape=jax.ShapeDtypeStruct((M_PER, N_OUT), jnp.bfloat16),
        in_specs=[
            pl.BlockSpec(memory_space=pltpu.ANY),
            pl.BlockSpec(memory_space=pltpu.VMEM),
        ],
        out_specs=pl.BlockSpec(memory_space=pltpu.VMEM),
        scratch_shapes=[
            pltpu.VMEM((MC, K), jnp.bfloat16),
            pltpu.VMEM((N_DEV - 1, MC, K), jnp.bfloat16),
            pltpu.VMEM((N_DEV - 1, MC, K), jnp.bfloat16),
            pltpu.SemaphoreType.DMA((N_DEV - 1,)),
            pltpu.SemaphoreType.DMA((N_DEV - 1,)),
            pltpu.SemaphoreType.DMA((N_DEV - 1,)),
            pltpu.SemaphoreType.DMA((N_DEV - 1,)),
            pltpu.SemaphoreType.DMA,
        ],
        compiler_params=pltpu.CompilerParams(collective_id=0),
    )(tb, wb)


# baseline (device time: 676374 ns/iter reference)
import jax
import jax.numpy as jnp
from jax import lax
from jax.experimental import pallas as pl
from jax.experimental.pallas import tpu as pltpu

N_DEV = 4
M_PER = 8192
MC = M_PER // N_DEV
K = 2048
N_OUT = 2048
RB = 512


def kernel(t, W):
    tb = t.astype(jnp.bfloat16)
    wb = W.astype(jnp.bfloat16)

    def body(t_hbm, w_ref, out_hbm, hbm_recv, va, vb, obuf,
             rs_send_sems, rs_recv_sems, ag_send_sems, ag_recv_sems,
             stream_sems, wb_sem, out_sems, credit_sem):
        my = lax.axis_index("i")
        right = jnp.mod(my + 1, N_DEV)
        left = jnp.mod(my + N_DEV - 1, N_DEV)

        barrier_sem = pltpu.get_barrier_semaphore()
        for nbr in (left, right):
            pl.semaphore_signal(
                barrier_sem, inc=1,
                device_id=(nbr,), device_id_type=pl.DeviceIdType.MESH,
            )
        pl.semaphore_wait(barrier_sem, 2)

        def t_rows(c, b):
            return t_hbm.at[pl.ds(c * MC + b * RB, RB)]

        for s in range(N_DEV - 1):
            if s == 0:
                src = t_hbm.at[pl.ds(my * MC, MC)]
            else:
                src = hbm_recv.at[(s - 1) % 2]
            if s == 2:
                pl.semaphore_wait(credit_sem, 1)
            rdma = pltpu.make_async_remote_copy(
                src_ref=src,
                dst_ref=hbm_recv.at[s % 2],
                send_sem=rs_send_sems.at[s],
                recv_sem=rs_recv_sems.at[s],
                device_id=(right,),
                device_id_type=pl.DeviceIdType.MESH,
            )
            rdma.start()
            rdma.wait()
            if s == 1:
                pl.semaphore_signal(
                    credit_sem, inc=1,
                    device_id=(left,), device_id_type=pl.DeviceIdType.MESH,
                )
            c_in = jnp.mod(my + N_DEV - s - 1, N_DEV)
            if s < N_DEV - 2:
                for b in range(MC // RB):
                    rslab = hbm_recv.at[s % 2, pl.ds(b * RB, RB)]
                    cpa = pltpu.make_async_copy(rslab, va, stream_sems.at[0])
                    cpb = pltpu.make_async_copy(t_rows(c_in, b), vb,
                                                stream_sems.at[1])
                    cpa.start()
                    cpb.start()
                    cpa.wait()
                    cpb.wait()
                    va[...] = va[...] + vb[...]
                    cpw = pltpu.make_async_copy(va, rslab, wb_sem)
                    cpw.start()
                    cpw.wait()
            else:
                own_c = jnp.mod(my + 1, N_DEV)
                for b in range(MC // RB):
                    oslot = b % 2
                    rslab = hbm_recv.at[s % 2, pl.ds(b * RB, RB)]
                    cpa = pltpu.make_async_copy(rslab, va, stream_sems.at[0])
                    cpb = pltpu.make_async_copy(t_rows(c_in, b), vb,
                                                stream_sems.at[1])
                    cpa.start()
                    cpb.start()
                    cpa.wait()
                    cpb.wait()
                    if b >= 2:
                        pltpu.make_async_copy(
                            obuf.at[oslot],
                            out_hbm.at[pl.ds(own_c * MC, RB)],
                            out_sems.at[oslot],
                        ).wait()
                    acc = jnp.dot(va[...] + vb[...], w_ref[...],
                                  preferred_element_type=jnp.float32)
                    obuf[oslot] = acc.astype(jnp.bfloat16)
                    pltpu.make_async_copy(
                        obuf.at[oslot],
                        out_hbm.at[pl.ds(own_c * MC + b * RB, RB)],
                        out_sems.at[oslot],
                    ).start()
                for oslot in range(2):
                    pltpu.make_async_copy(
                        obuf.at[oslot],
                        out_hbm.at[pl.ds(own_c * MC, RB)],
                        out_sems.at[oslot],
                    ).wait()

        for h in range(N_DEV - 1):
            c_send = jnp.mod(my + 1 - h + N_DEV, N_DEV)
            rdma = pltpu.make_async_remote_copy(
                src_ref=out_hbm.at[pl.ds(c_send * MC, MC)],
                dst_ref=out_hbm.at[pl.ds(c_send * MC, MC)],
                send_sem=ag_send_sems.at[h],
                recv_sem=ag_recv_sems.at[h],
                device_id=(right,),
                device_id_type=pl.DeviceIdType.MESH,
            )
            rdma.start()
            rdma.wait()

    out, _ = pl.pallas_call(
        body,
        out_shape=(
            jax.ShapeDtypeStruct((M_PER, N_OUT), jnp.bfloat16),
            jax.ShapeDtypeStruct((2, MC, K), jnp.bfloat16),
        ),
        in_specs=[
            pl.BlockSpec(memory_space=pl.ANY),
            pl.BlockSpec(memory_space=pltpu.VMEM),
        ],
        out_specs=(
            pl.BlockSpec(memory_space=pl.ANY),
            pl.BlockSpec(memory_space=pl.ANY),
        ),
        scratch_shapes=[
            pltpu.VMEM((RB, K), jnp.bfloat16),
            pltpu.VMEM((RB, K), jnp.bfloat16),
            pltpu.VMEM((2, RB, N_OUT), jnp.bfloat16),
            pltpu.SemaphoreType.DMA((N_DEV - 1,)),
            pltpu.SemaphoreType.DMA((N_DEV - 1,)),
            pltpu.SemaphoreType.DMA((N_DEV - 1,)),
            pltpu.SemaphoreType.DMA((N_DEV - 1,)),
            pltpu.SemaphoreType.DMA((2,)),
            pltpu.SemaphoreType.DMA,
            pltpu.SemaphoreType.DMA((2,)),
            pltpu.SemaphoreType.REGULAR,
        ],
        compiler_params=pltpu.CompilerParams(collective_id=0),
    )(tb, wb)
    return out


# device time: 397300 ns/iter; 1.7024x vs baseline; 1.7024x over previous
import jax
import jax.numpy as jnp
from jax import lax
from jax.experimental import pallas as pl
from jax.experimental.pallas import tpu as pltpu

N_DEV = 4
M_PER = 8192
MC = M_PER // N_DEV
HC = MC // 2
K = 2048
N_OUT = 2048
RB = 512


def kernel(t, W):
    tb = t.astype(jnp.bfloat16)
    wb = W.astype(jnp.bfloat16)

    def body(t_hbm, w_ref, out_hbm, hbm_recv, va, vb, obuf,
             rs_send_sems, rs_recv_sems, ag_send_sems, ag_recv_sems,
             stream_sems, wb_sem, out_sems, credit_p, credit_m):
        my = lax.axis_index("i")
        right = jnp.mod(my + 1, N_DEV)
        left = jnp.mod(my + N_DEV - 1, N_DEV)

        barrier_sem = pltpu.get_barrier_semaphore()
        for nbr in (left, right):
            pl.semaphore_signal(
                barrier_sem, inc=1,
                device_id=(nbr,), device_id_type=pl.DeviceIdType.MESH,
            )
        pl.semaphore_wait(barrier_sem, 2)

        def t_half(c, d):
            return t_hbm.at[pl.ds(c * MC + d * HC, HC)]

        def rmw_add(recv_ref, own_ref):
            cpa = pltpu.make_async_copy(recv_ref, va, stream_sems.at[0])
            cpb = pltpu.make_async_copy(own_ref, vb, stream_sems.at[1])
            cpa.start()
            cpb.start()
            cpa.wait()
            cpb.wait()
            va[...] = va[...] + vb[...]
            cpw = pltpu.make_async_copy(va, recv_ref, wb_sem)
            cpw.start()
            cpw.wait()

        for s in range(N_DEV - 1):
            slot = s % 2
            if s == 0:
                src_p = t_half(my, 0)
                src_m = t_half(my, 1)
            else:
                src_p = hbm_recv.at[0, (s - 1) % 2]
                src_m = hbm_recv.at[1, (s - 1) % 2]
            if s == 2:
                pl.semaphore_wait(credit_p, 1)
                pl.semaphore_wait(credit_m, 1)
            rdma_p = pltpu.make_async_remote_copy(
                src_ref=src_p,
                dst_ref=hbm_recv.at[0, slot],
                send_sem=rs_send_sems.at[0, s],
                recv_sem=rs_recv_sems.at[0, s],
                device_id=(right,),
                device_id_type=pl.DeviceIdType.MESH,
            )
            rdma_m = pltpu.make_async_remote_copy(
                src_ref=src_m,
                dst_ref=hbm_recv.at[1, slot],
                send_sem=rs_send_sems.at[1, s],
                recv_sem=rs_recv_sems.at[1, s],
                device_id=(left,),
                device_id_type=pl.DeviceIdType.MESH,
            )
            rdma_p.start()
            rdma_m.start()
            rdma_p.wait()
            rdma_m.wait()
            if s == 1:
                pl.semaphore_signal(
                    credit_p, inc=1,
                    device_id=(left,), device_id_type=pl.DeviceIdType.MESH,
                )
                pl.semaphore_signal(
                    credit_m, inc=1,
                    device_id=(right,), device_id_type=pl.DeviceIdType.MESH,
                )
            cin_p = jnp.mod(my + N_DEV - s - 1, N_DEV)
            cin_m = jnp.mod(my + s + 1, N_DEV)
            if s < N_DEV - 2:
                rmw_add(hbm_recv.at[0, slot], t_half(cin_p, 0))
                rmw_add(hbm_recv.at[1, slot], t_half(cin_m, 1))
            else:
                own = ((jnp.mod(my + 1, N_DEV), 0),
                       (jnp.mod(my + N_DEV - 1, N_DEV), 1))
                j = 0
                for d, (own_c, dd) in enumerate(own):
                    cin = cin_p if d == 0 else cin_m
                    cpa = pltpu.make_async_copy(
                        hbm_recv.at[d, slot], va, stream_sems.at[0])
                    cpb = pltpu.make_async_copy(
                        t_half(cin, dd), vb, stream_sems.at[1])
                    cpa.start()
                    cpb.start()
                    cpa.wait()
                    cpb.wait()
                    for b in range(HC // RB):
                        oslot = j % 2
                        if j >= 2:
                            pltpu.make_async_copy(
                                obuf.at[oslot],
                                out_hbm.at[pl.ds(own_c * MC, RB)],
                                out_sems.at[oslot],
                            ).wait()
                        acc = jnp.dot(
                            va[pl.ds(b * RB, RB)] + vb[pl.ds(b * RB, RB)],
                            w_ref[...],
                            preferred_element_type=jnp.float32)
                        obuf[oslot] = acc.astype(jnp.bfloat16)
                        pltpu.make_async_copy(
                            obuf.at[oslot],
                            out_hbm.at[
                                pl.ds(own_c * MC + dd * HC + b * RB, RB)],
                            out_sems.at[oslot],
                        ).start()
                        j += 1
                for oslot in range(2):
                    pltpu.make_async_copy(
                        obuf.at[oslot],
                        out_hbm.at[pl.ds(0, RB)],
                        out_sems.at[oslot],
                    ).wait()

        for h in range(N_DEV - 1):
            cs_p = jnp.mod(my + 1 - h + N_DEV, N_DEV)
            cs_m = jnp.mod(my - 1 + h + N_DEV, N_DEV)
            rows_p = out_hbm.at[pl.ds(cs_p * MC, HC)]
            rows_m = out_hbm.at[pl.ds(cs_m * MC + HC, HC)]
            rdma_p = pltpu.make_async_remote_copy(
                src_ref=rows_p, dst_ref=rows_p,
                send_sem=ag_send_sems.at[0, h],
                recv_sem=ag_recv_sems.at[0, h],
                device_id=(right,),
                device_id_type=pl.DeviceIdType.MESH,
            )
            rdma_m = pltpu.make_async_remote_copy(
                src_ref=rows_m, dst_ref=rows_m,
                send_sem=ag_send_sems.at[1, h],
                recv_sem=ag_recv_sems.at[1, h],
                device_id=(left,),
                device_id_type=pl.DeviceIdType.MESH,
            )
            rdma_p.start()
            rdma_m.start()
            rdma_p.wait()
            rdma_m.wait()

    out, _ = pl.pallas_call(
        body,
        out_shape=(
            jax.ShapeDtypeStruct((M_PER, N_OUT), jnp.bfloat16),
            jax.ShapeDtypeStruct((2, 2, HC, K), jnp.bfloat16),
        ),
        in_specs=[
            pl.BlockSpec(memory_space=pl.ANY),
            pl.BlockSpec(memory_space=pltpu.VMEM),
        ],
        out_specs=(
            pl.BlockSpec(memory_space=pl.ANY),
            pl.BlockSpec(memory_space=pl.ANY),
        ),
        scratch_shapes=[
            pltpu.VMEM((HC, K), jnp.bfloat16),
            pltpu.VMEM((HC, K), jnp.bfloat16),
            pltpu.VMEM((2, RB, N_OUT), jnp.bfloat16),
            pltpu.SemaphoreType.DMA((2, N_DEV - 1)),
            pltpu.SemaphoreType.DMA((2, N_DEV - 1)),
            pltpu.SemaphoreType.DMA((2, N_DEV - 1)),
            pltpu.SemaphoreType.DMA((2, N_DEV - 1)),
            pltpu.SemaphoreType.DMA((2,)),
            pltpu.SemaphoreType.DMA,
            pltpu.SemaphoreType.DMA((2,)),
            pltpu.SemaphoreType.REGULAR,
            pltpu.SemaphoreType.REGULAR,
        ],
        compiler_params=pltpu.CompilerParams(collective_id=0),
    )(tb, wb)
    return out


# device time: 369516 ns/iter; 1.8304x vs baseline; 1.0752x over previous
import jax
import jax.numpy as jnp
from jax import lax
from jax.experimental import pallas as pl
from jax.experimental.pallas import tpu as pltpu

N_DEV = 4
M_PER = 8192
MC = M_PER // N_DEV
HC = MC // 2
K = 2048
N_OUT = 2048
RB = 256


def kernel(t, W):
    tb = t.astype(jnp.bfloat16)
    wb = W.astype(jnp.bfloat16)

    def body(t_hbm, w_ref, out_hbm, rv, vo, obuf,
             rs_send_sems, rs_recv_sems, ag_send_sems, ag_recv_sems,
             own_sems, out_sems, credit_p, credit_m):
        my = lax.axis_index("i")
        right = jnp.mod(my + 1, N_DEV)
        left = jnp.mod(my + N_DEV - 1, N_DEV)

        barrier_sem = pltpu.get_barrier_semaphore()
        for nbr in (left, right):
            pl.semaphore_signal(
                barrier_sem, inc=1,
                device_id=(nbr,), device_id_type=pl.DeviceIdType.MESH,
            )
        pl.semaphore_wait(barrier_sem, 2)

        def t_half(c, d):
            return t_hbm.at[pl.ds(c * MC + d * HC, HC)]

        for s in range(N_DEV - 1):
            slot = s % 2
            if s == 0:
                src_p = t_half(my, 0)
                src_m = t_half(my, 1)
            else:
                src_p = rv.at[0, (s - 1) % 2]
                src_m = rv.at[1, (s - 1) % 2]
            if s == 2:
                pl.semaphore_wait(credit_p, 1)
                pl.semaphore_wait(credit_m, 1)
            rdma_p = pltpu.make_async_remote_copy(
                src_ref=src_p,
                dst_ref=rv.at[0, slot],
                send_sem=rs_send_sems.at[0, s],
                recv_sem=rs_recv_sems.at[0, s],
                device_id=(right,),
                device_id_type=pl.DeviceIdType.MESH,
            )
            rdma_m = pltpu.make_async_remote_copy(
                src_ref=src_m,
                dst_ref=rv.at[1, slot],
                send_sem=rs_send_sems.at[1, s],
                recv_sem=rs_recv_sems.at[1, s],
                device_id=(left,),
                device_id_type=pl.DeviceIdType.MESH,
            )
            rdma_p.start()
            rdma_m.start()
            cin_p = jnp.mod(my + N_DEV - s - 1, N_DEV)
            cin_m = jnp.mod(my + s + 1, N_DEV)
            cp_p = pltpu.make_async_copy(t_half(cin_p, 0), vo.at[0],
                                         own_sems.at[0])
            cp_m = pltpu.make_async_copy(t_half(cin_m, 1), vo.at[1],
                                         own_sems.at[1])
            cp_p.start()
            cp_m.start()
            cp_p.wait()
            cp_m.wait()
            rdma_p.wait()
            rdma_m.wait()
            if s == 1:
                pl.semaphore_signal(
                    credit_p, inc=1,
                    device_id=(left,), device_id_type=pl.DeviceIdType.MESH,
                )
                pl.semaphore_signal(
                    credit_m, inc=1,
                    device_id=(right,), device_id_type=pl.DeviceIdType.MESH,
                )
            if s < N_DEV - 2:
                rv[0, slot] = rv[0, slot] + vo[0]
                rv[1, slot] = rv[1, slot] + vo[1]
            else:
                own = ((jnp.mod(my + 1, N_DEV), 0),
                       (jnp.mod(my + N_DEV - 1, N_DEV), 1))
                j = 0
                for d, (own_c, dd) in enumerate(own):
                    for b in range(HC // RB):
                        oslot = j % 2
                        if j >= 2:
                            pltpu.make_async_copy(
                                obuf.at[oslot],
                                out_hbm.at[pl.ds(own_c * MC, RB)],
                                out_sems.at[oslot],
                            ).wait()
                        blk = (rv[d, slot, pl.ds(b * RB, RB)]
                               + vo[d, pl.ds(b * RB, RB)])
                        acc = jnp.dot(blk, w_ref[...],
                                      preferred_element_type=jnp.float32)
                        obuf[oslot] = acc.astype(jnp.bfloat16)
                        pltpu.make_async_copy(
                            obuf.at[oslot],
                            out_hbm.at[
                                pl.ds(own_c * MC + dd * HC + b * RB, RB)],
                            out_sems.at[oslot],
                        ).start()
                        j += 1
                for oslot in range(2):
                    pltpu.make_async_copy(
                        obuf.at[oslot],
                        out_hbm.at[pl.ds(0, RB)],
                        out_sems.at[oslot],
                    ).wait()

        for h in range(N_DEV - 1):
            cs_p = jnp.mod(my + 1 - h + N_DEV, N_DEV)
            cs_m = jnp.mod(my - 1 + h + N_DEV, N_DEV)
            rows_p = out_hbm.at[pl.ds(cs_p * MC, HC)]
            rows_m = out_hbm.at[pl.ds(cs_m * MC + HC, HC)]
            rdma_p = pltpu.make_async_remote_copy(
                src_ref=rows_p, dst_ref=rows_p,
                send_sem=ag_send_sems.at[0, h],
                recv_sem=ag_recv_sems.at[0, h],
                device_id=(right,),
                device_id_type=pl.DeviceIdType.MESH,
            )
            rdma_m = pltpu.make_async_remote_copy(
                src_ref=rows_m, dst_ref=rows_m,
                send_sem=ag_send_sems.at[1, h],
                recv_sem=ag_recv_sems.at[1, h],
                device_id=(left,),
                device_id_type=pl.DeviceIdType.MESH,
            )
            rdma_p.start()
            rdma_m.start()
            rdma_p.wait()
            rdma_m.wait()

    return pl.pallas_call(
        body,
        out_shape=jax.ShapeDtypeStruct((M_PER, N_OUT), jnp.bfloat16),
        in_specs=[
            pl.BlockSpec(memory_space=pl.ANY),
            pl.BlockSpec(memory_space=pltpu.VMEM),
        ],
        out_specs=pl.BlockSpec(memory_space=pl.ANY),
        scratch_shapes=[
            pltpu.VMEM((2, 2, HC, K), jnp.bfloat16),
            pltpu.VMEM((2, HC, K), jnp.bfloat16),
            pltpu.VMEM((2, RB, N_OUT), jnp.bfloat16),
            pltpu.SemaphoreType.DMA((2, N_DEV - 1)),
            pltpu.SemaphoreType.DMA((2, N_DEV - 1)),
            pltpu.SemaphoreType.DMA((2, N_DEV - 1)),
            pltpu.SemaphoreType.DMA((2, N_DEV - 1)),
            pltpu.SemaphoreType.DMA((2,)),
            pltpu.SemaphoreType.DMA((2,)),
            pltpu.SemaphoreType.REGULAR,
            pltpu.SemaphoreType.REGULAR,
        ],
        compiler_params=pltpu.CompilerParams(collective_id=0),
    )(tb, wb)


# device time: 337458 ns/iter; 2.0043x vs baseline; 1.0950x over previous
import jax
import jax.numpy as jnp
from jax import lax
from jax.experimental import pallas as pl
from jax.experimental.pallas import tpu as pltpu

N_DEV = 4
M_PER = 8192
MC = M_PER // N_DEV
HC = MC // 2
SB = HC // 2
K = 2048
N_OUT = 2048
RB = 256


def kernel(t, W):
    tb = t.astype(jnp.bfloat16)
    wb = W.astype(jnp.bfloat16)

    def body(t_hbm, w_ref, out_hbm, rv, vo, obuf,
             rs_send_sems, rs_recv_sems, ag_send_sems, ag_recv_sems,
             own_sems, out_sems, credit_p0, credit_p1, credit_m0,
             credit_m1):
        my = lax.axis_index("i")
        right = jnp.mod(my + 1, N_DEV)
        left = jnp.mod(my + N_DEV - 1, N_DEV)
        credit = {(0, 0): credit_p0, (0, 1): credit_p1,
                  (1, 0): credit_m0, (1, 1): credit_m1}
        peer = {0: (right, left), 1: (left, right)}

        barrier_sem = pltpu.get_barrier_semaphore()
        for nbr in (left, right):
            pl.semaphore_signal(
                barrier_sem, inc=1,
                device_id=(nbr,), device_id_type=pl.DeviceIdType.MESH,
            )
        pl.semaphore_wait(barrier_sem, 2)

        def t_rows(c, d, q):
            return t_hbm.at[pl.ds(c * MC + d * HC + q * SB, SB)]

        def cin(d, s):
            return (jnp.mod(my + N_DEV - s - 1, N_DEV) if d == 0
                    else jnp.mod(my + s + 1, N_DEV))

        def rs_rdma(d, s, q):
            if s == 0:
                c0 = my if d == 0 else my
                src = t_rows(c0, d, q)
            else:
                src = rv.at[d, (s - 1) % 2, pl.ds(q * SB, SB)]
            return pltpu.make_async_remote_copy(
                src_ref=src,
                dst_ref=rv.at[d, s % 2, pl.ds(q * SB, SB)],
                send_sem=rs_send_sems.at[d, s, q],
                recv_sem=rs_recv_sems.at[d, s, q],
                device_id=(peer[d][0],),
                device_id_type=pl.DeviceIdType.MESH,
            )

        def start_vo_load(s):
            cps = []
            for d in (0, 1):
                cp = pltpu.make_async_copy(
                    t_hbm.at[pl.ds(cin(d, s) * MC + d * HC, HC)],
                    vo.at[d], own_sems.at[d])
                cp.start()
                cps.append(cp)
            return cps

        def add_own(s, q):
            for d in (0, 1):
                rows = pl.ds(q * SB, SB)
                rv[d, s % 2, rows] = rv[d, s % 2, rows] + vo[d, rows]

        own_c = (jnp.mod(my + 1, N_DEV), jnp.mod(my + N_DEV - 1, N_DEV))

        def matmul_sub(q):
            j = 0
            for d in (0, 1):
                for b in range(SB // RB):
                    oslot = j % 2
                    if j >= 2:
                        pltpu.make_async_copy(
                            obuf.at[oslot], out_hbm.at[pl.ds(0, RB)],
                            out_sems.at[oslot]).wait()
                    rows = pl.ds(q * SB + b * RB, RB)
                    acc = jnp.dot(rv[d, 0, rows] + vo[d, rows], w_ref[...],
                                  preferred_element_type=jnp.float32)
                    obuf[oslot] = acc.astype(jnp.bfloat16)
                    pltpu.make_async_copy(
                        obuf.at[oslot],
                        out_hbm.at[pl.ds(
                            own_c[d] * MC + d * HC + q * SB + b * RB, RB)],
                        out_sems.at[oslot]).start()
                    j += 1
            for oslot in range(2):
                pltpu.make_async_copy(
                    obuf.at[oslot], out_hbm.at[pl.ds(0, RB)],
                    out_sems.at[oslot]).wait()

        def ag_rdma(h, q):
            rs = []
            for d in (0, 1):
                cs = (jnp.mod(my + 1 - h + N_DEV, N_DEV) if d == 0
                      else jnp.mod(my - 1 + h + N_DEV, N_DEV))
                rows = out_hbm.at[pl.ds(cs * MC + d * HC + q * SB, SB)]
                rs.append(pltpu.make_async_remote_copy(
                    src_ref=rows, dst_ref=rows,
                    send_sem=ag_send_sems.at[d, h, q],
                    recv_sem=ag_recv_sems.at[d, h, q],
                    device_id=(peer[d][0],),
                    device_id_type=pl.DeviceIdType.MESH,
                ))
            return rs

        rs0 = {q: [rs_rdma(d, 0, q) for d in (0, 1)] for q in (0, 1)}
        for q in (0, 1):
            for r in rs0[q]:
                r.start()
        vo_cps = start_vo_load(0)
        for cp in vo_cps:
            cp.wait()

        rs_next = {}
        for q in (0, 1):
            for r in rs0[q]:
                r.wait()
            add_own(0, q)
            rs_next[q] = [rs_rdma(d, 1, q) for d in (0, 1)]
            for r in rs_next[q]:
                r.start()

        vo_cps = start_vo_load(1)
        for cp in vo_cps:
            cp.wait()

        rs_last = {}
        for q in (0, 1):
            for r in rs_next[q]:
                r.wait()
            add_own(1, q)
            for d in (0, 1):
                pl.semaphore_signal(
                    credit[(d, q)], inc=1,
                    device_id=(peer[d][1],),
                    device_id_type=pl.DeviceIdType.MESH,
                )
            for d in (0, 1):
                pl.semaphore_wait(credit[(d, q)], 1)
            rs_last[q] = [rs_rdma(d, 2, q) for d in (0, 1)]
            for r in rs_last[q]:
                r.start()

        vo_cps = start_vo_load(2)
        for cp in vo_cps:
            cp.wait()

        ag0 = {}
        for q in (0, 1):
            for r in rs_last[q]:
                r.wait()
            matmul_sub(q)
            ag0[q] = ag_rdma(0, q)
            for r in ag0[q]:
                r.start()

        ag_prev = ag0
        for h in (1, 2):
            ag_h = {}
            for q in (0, 1):
                for r in ag_prev[q]:
                    r.wait()
                ag_h[q] = ag_rdma(h, q)
                for r in ag_h[q]:
                    r.start()
            ag_prev = ag_h
        for q in (0, 1):
            for r in ag_prev[q]:
                r.wait()

    return pl.pallas_call(
        body,
        out_shape=jax.ShapeDtypeStruct((M_PER, N_OUT), jnp.bfloat16),
        in_specs=[
            pl.BlockSpec(memory_space=pl.ANY),
            pl.BlockSpec(memory_space=pltpu.VMEM),
        ],
        out_specs=pl.BlockSpec(memory_space=pl.ANY),
        scratch_shapes=[
            pltpu.VMEM((2, 2, HC, K), jnp.bfloat16),
            pltpu.VMEM((2, HC, K), jnp.bfloat16),
            pltpu.VMEM((2, RB, N_OUT), jnp.bfloat16),
            pltpu.SemaphoreType.DMA((2, N_DEV - 1, 2)),
            pltpu.SemaphoreType.DMA((2, N_DEV - 1, 2)),
            pltpu.SemaphoreType.DMA((2, N_DEV - 1, 2)),
            pltpu.SemaphoreType.DMA((2, N_DEV - 1, 2)),
            pltpu.SemaphoreType.DMA((2,)),
            pltpu.SemaphoreType.DMA((2,)),
            pltpu.SemaphoreType.REGULAR,
            pltpu.SemaphoreType.REGULAR,
            pltpu.SemaphoreType.REGULAR,
            pltpu.SemaphoreType.REGULAR,
        ],
        compiler_params=pltpu.CompilerParams(collective_id=0),
    )(tb, wb)


# device time: 316555 ns/iter; 2.1367x vs baseline; 1.0660x over previous
import jax
import jax.numpy as jnp
from jax import lax
from jax.experimental import pallas as pl
from jax.experimental.pallas import tpu as pltpu

N_DEV = 4
M_PER = 8192
MC = M_PER // N_DEV
HC = MC // 2
SB = HC // 2
K = 2048
N_OUT = 2048
RB = 256


def kernel(t, W):
    wb = W.astype(jnp.bfloat16)

    def body(t_hbm, w_ref, out_hbm, rv, vo, obuf,
             rs_send_sems, rs_recv_sems, ag_send_sems, ag_recv_sems,
             own_sems, out_sems, credit_p0, credit_p1, credit_m0,
             credit_m1):
        my = lax.axis_index("i")
        right = jnp.mod(my + 1, N_DEV)
        left = jnp.mod(my + N_DEV - 1, N_DEV)
        credit = {(0, 0): credit_p0, (0, 1): credit_p1,
                  (1, 0): credit_m0, (1, 1): credit_m1}
        peer = {0: (right, left), 1: (left, right)}

        barrier_sem = pltpu.get_barrier_semaphore()
        for nbr in (left, right):
            pl.semaphore_signal(
                barrier_sem, inc=1,
                device_id=(nbr,), device_id_type=pl.DeviceIdType.MESH,
            )
        pl.semaphore_wait(barrier_sem, 2)

        def cin(d, s):
            return (jnp.mod(my + N_DEV - s - 1, N_DEV) if d == 0
                    else jnp.mod(my + s + 1, N_DEV))

        def vo_load(c_of_d, q):
            cps = []
            for d in (0, 1):
                cp = pltpu.make_async_copy(
                    t_hbm.at[pl.ds(c_of_d(d) * MC + d * HC + q * SB, SB)],
                    vo.at[d], own_sems.at[d])
                cp.start()
                cps.append(cp)
            return cps

        def rs_rdma(d, s, q):
            if s == 0:
                src = out_hbm.at[pl.ds(my * MC + d * HC + q * SB, SB)]
            else:
                src = rv.at[d, (s - 1) % 2, pl.ds(q * SB, SB)]
            return pltpu.make_async_remote_copy(
                src_ref=src,
                dst_ref=rv.at[d, s % 2, pl.ds(q * SB, SB)],
                send_sem=rs_send_sems.at[d, s, q],
                recv_sem=rs_recv_sems.at[d, s, q],
                device_id=(peer[d][0],),
                device_id_type=pl.DeviceIdType.MESH,
            )

        own_c = (jnp.mod(my + 1, N_DEV), jnp.mod(my + N_DEV - 1, N_DEV))

        def matmul_sub(q):
            j = 0
            for d in (0, 1):
                for b in range(SB // RB):
                    oslot = j % 2
                    if j >= 2:
                        pltpu.make_async_copy(
                            obuf.at[oslot], out_hbm.at[pl.ds(0, RB)],
                            out_sems.at[oslot]).wait()
                    rows = pl.ds(q * SB + b * RB, RB)
                    vrows = pl.ds(b * RB, RB)
                    blk = (rv[d, 0, rows]
                           + vo[d, vrows].astype(jnp.bfloat16))
                    acc = jnp.dot(blk, w_ref[...],
                                  preferred_element_type=jnp.float32)
                    obuf[oslot] = acc.astype(jnp.bfloat16)
                    pltpu.make_async_copy(
                        obuf.at[oslot],
                        out_hbm.at[pl.ds(
                            own_c[d] * MC + d * HC + q * SB + b * RB, RB)],
                        out_sems.at[oslot]).start()
                    j += 1
            for oslot in range(2):
                pltpu.make_async_copy(
                    obuf.at[oslot], out_hbm.at[pl.ds(0, RB)],
                    out_sems.at[oslot]).wait()

        def ag_rdma(h, q):
            rs = []
            for d in (0, 1):
                cs = (jnp.mod(my + 1 - h + N_DEV, N_DEV) if d == 0
                      else jnp.mod(my - 1 + h + N_DEV, N_DEV))
                rows = out_hbm.at[pl.ds(cs * MC + d * HC + q * SB, SB)]
                rs.append(pltpu.make_async_remote_copy(
                    src_ref=rows, dst_ref=rows,
                    send_sem=ag_send_sems.at[d, h, q],
                    recv_sem=ag_recv_sems.at[d, h, q],
                    device_id=(peer[d][0],),
                    device_id_type=pl.DeviceIdType.MESH,
                ))
            return rs

        rs_cur = {}
        for q in (0, 1):
            cps = vo_load(lambda d: my, q)
            for cp in cps:
                cp.wait()
            j = 0
            for d in (0, 1):
                for b in range(SB // RB):
                    oslot = j % 2
                    if j >= 2:
                        pltpu.make_async_copy(
                            obuf.at[oslot], out_hbm.at[pl.ds(0, RB)],
                            out_sems.at[oslot]).wait()
                    obuf[oslot] = vo[d, pl.ds(b * RB, RB)].astype(
                        jnp.bfloat16)
                    pltpu.make_async_copy(
                        obuf.at[oslot],
                        out_hbm.at[pl.ds(
                            my * MC + d * HC + q * SB + b * RB, RB)],
                        out_sems.at[oslot]).start()
                    j += 1
            for oslot in range(2):
                pltpu.make_async_copy(
                    obuf.at[oslot], out_hbm.at[pl.ds(0, RB)],
                    out_sems.at[oslot]).wait()
            rs_cur[q] = [rs_rdma(d, 0, q) for d in (0, 1)]
            for r in rs_cur[q]:
                r.start()

        pending = vo_load(lambda d: cin(d, 0), 0)

        for s in (0, 1):
            rs_next = {}
            for q in (0, 1):
                for cp in pending:
                    cp.wait()
                for r in rs_cur[q]:
                    r.wait()
                for d in (0, 1):
                    rows = pl.ds(q * SB, SB)
                    rv[d, s % 2, rows] = (rv[d, s % 2, rows]
                                          + vo[d].astype(jnp.bfloat16))
                pending = (vo_load(lambda d: cin(d, s), 1) if q == 0
                           else vo_load(lambda d: cin(d, s + 1), 0))
                if s == 1:
                    for d in (0, 1):
                        pl.semaphore_signal(
                            credit[(d, q)], inc=1,
                            device_id=(peer[d][1],),
                            device_id_type=pl.DeviceIdType.MESH,
                        )
                    for d in (0, 1):
                        pl.semaphore_wait(credit[(d, q)], 1)
                rs_next[q] = [rs_rdma(d, s + 1, q) for d in (0, 1)]
                for r in rs_next[q]:
                    r.start()
            rs_cur = rs_next

        ag_prev = {}
        for q in (0, 1):
            for cp in pending:
                cp.wait()
            for r in rs_cur[q]:
                r.wait()
            matmul_sub(q)
            if q == 0:
                pending = vo_load(lambda d: cin(d, 2), 1)
            ag_prev[q] = ag_rdma(0, q)
            for r in ag_prev[q]:
                r.start()

        for h in (1, 2):
            ag_h = {}
            for q in (0, 1):
                for r in ag_prev[q]:
                    r.wait()
                ag_h[q] = ag_rdma(h, q)
                for r in ag_h[q]:
                    r.start()
            ag_prev = ag_h
        for q in (0, 1):
            for r in ag_prev[q]:
                r.wait()

    return pl.pallas_call(
        body,
        out_shape=jax.ShapeDtypeStruct((M_PER, N_OUT), jnp.bfloat16),
        in_specs=[
            pl.BlockSpec(memory_space=pl.ANY),
            pl.BlockSpec(memory_space=pltpu.VMEM),
        ],
        out_specs=pl.BlockSpec(memory_space=pl.ANY),
        scratch_shapes=[
            pltpu.VMEM((2, 2, HC, K), jnp.bfloat16),
            pltpu.VMEM((2, SB, K), jnp.float32),
            pltpu.VMEM((2, RB, N_OUT), jnp.bfloat16),
            pltpu.SemaphoreType.DMA((2, N_DEV - 1, 2)),
            pltpu.SemaphoreType.DMA((2, N_DEV - 1, 2)),
            pltpu.SemaphoreType.DMA((2, N_DEV - 1, 2)),
            pltpu.SemaphoreType.DMA((2, N_DEV - 1, 2)),
            pltpu.SemaphoreType.DMA((2,)),
            pltpu.SemaphoreType.DMA((2,)),
            pltpu.SemaphoreType.REGULAR,
            pltpu.SemaphoreType.REGULAR,
            pltpu.SemaphoreType.REGULAR,
            pltpu.SemaphoreType.REGULAR,
        ],
        compiler_params=pltpu.CompilerParams(
            collective_id=0,
            vmem_limit_bytes=48 * 1024 * 1024,
        ),
    )(t, wb)


# device time: 315299 ns/iter; 2.1452x vs baseline; 1.0040x over previous
import jax
import jax.numpy as jnp
from jax import lax
from jax.experimental import pallas as pl
from jax.experimental.pallas import tpu as pltpu

N_DEV = 4
M_PER = 8192
MC = M_PER // N_DEV
HC = MC // 2
SB = HC // 2
K = 2048
N_OUT = 2048
RB = 256


def kernel(t, W):
    wb = W.astype(jnp.bfloat16)

    def body(t_hbm, w_ref, out_hbm, rv, vo, obuf,
             rs_send_sems, rs_recv_sems, ag_send_sems, ag_recv_sems,
             own_sems, out_sems, credit_p0, credit_p1, credit_m0,
             credit_m1):
        my = lax.axis_index("i")
        right = jnp.mod(my + 1, N_DEV)
        left = jnp.mod(my + N_DEV - 1, N_DEV)
        credit = {(0, 0): credit_p0, (0, 1): credit_p1,
                  (1, 0): credit_m0, (1, 1): credit_m1}
        peer = {0: (right, left), 1: (left, right)}

        barrier_sem = pltpu.get_barrier_semaphore()
        for nbr in (left, right):
            pl.semaphore_signal(
                barrier_sem, inc=1,
                device_id=(nbr,), device_id_type=pl.DeviceIdType.MESH,
            )

        def cin(d, s):
            return (jnp.mod(my + N_DEV - s - 1, N_DEV) if d == 0
                    else jnp.mod(my + s + 1, N_DEV))

        def vo_load(c_of_d, q):
            cps = []
            for d in (0, 1):
                cp = pltpu.make_async_copy(
                    t_hbm.at[pl.ds(c_of_d(d) * MC + d * HC + q * SB, SB)],
                    vo.at[d], own_sems.at[d])
                cp.start()
                cps.append(cp)
            return cps

        def rs_rdma(d, s, q):
            if s == 0:
                src = out_hbm.at[pl.ds(my * MC + d * HC + q * SB, SB)]
            else:
                src = rv.at[d, (s - 1) % 2, pl.ds(q * SB, SB)]
            return pltpu.make_async_remote_copy(
                src_ref=src,
                dst_ref=rv.at[d, s % 2, pl.ds(q * SB, SB)],
                send_sem=rs_send_sems.at[d, s, q],
                recv_sem=rs_recv_sems.at[d, s, q],
                device_id=(peer[d][0],),
                device_id_type=pl.DeviceIdType.MESH,
            )

        own_c = (jnp.mod(my + 1, N_DEV), jnp.mod(my + N_DEV - 1, N_DEV))

        def ag_rdma_dir(d, h, q):
            cs = (jnp.mod(my + 1 - h + N_DEV, N_DEV) if d == 0
                  else jnp.mod(my - 1 + h + N_DEV, N_DEV))
            rows = out_hbm.at[pl.ds(cs * MC + d * HC + q * SB, SB)]
            return pltpu.make_async_remote_copy(
                src_ref=rows, dst_ref=rows,
                send_sem=ag_send_sems.at[d, h, q],
                recv_sem=ag_recv_sems.at[d, h, q],
                device_id=(peer[d][0],),
                device_id_type=pl.DeviceIdType.MESH,
            )

        def ag_rdma(h, q):
            return [ag_rdma_dir(d, h, q) for d in (0, 1)]

        def matmul_sub(q):
            ags = []
            for d in (0, 1):
                for b in range(SB // RB):
                    rows = pl.ds(q * SB + b * RB, RB)
                    vrows = pl.ds(b * RB, RB)
                    blk = (rv[d, 0, rows]
                           + vo[d, vrows].astype(jnp.bfloat16))
                    acc = jnp.dot(blk, w_ref[...],
                                  preferred_element_type=jnp.float32)
                    obuf[b] = acc.astype(jnp.bfloat16)
                    pltpu.make_async_copy(
                        obuf.at[b],
                        out_hbm.at[pl.ds(
                            own_c[d] * MC + d * HC + q * SB + b * RB, RB)],
                        out_sems.at[b]).start()
                for b in range(SB // RB):
                    pltpu.make_async_copy(
                        obuf.at[b], out_hbm.at[pl.ds(0, RB)],
                        out_sems.at[b]).wait()
                ag = ag_rdma_dir(d, 0, q)
                ag.start()
                ags.append(ag)
            return ags

        rs_cur = {}
        for q in (0, 1):
            cps = vo_load(lambda d: my, q)
            for cp in cps:
                cp.wait()
            j = 0
            for d in (0, 1):
                for b in range(SB // RB):
                    oslot = j % 2
                    if j >= 2:
                        pltpu.make_async_copy(
                            obuf.at[oslot], out_hbm.at[pl.ds(0, RB)],
                            out_sems.at[oslot]).wait()
                    obuf[oslot] = vo[d, pl.ds(b * RB, RB)].astype(
                        jnp.bfloat16)
                    pltpu.make_async_copy(
                        obuf.at[oslot],
                        out_hbm.at[pl.ds(
                            my * MC + d * HC + q * SB + b * RB, RB)],
                        out_sems.at[oslot]).start()
                    j += 1
            for oslot in range(2):
                pltpu.make_async_copy(
                    obuf.at[oslot], out_hbm.at[pl.ds(0, RB)],
                    out_sems.at[oslot]).wait()
            if q == 0:
                pl.semaphore_wait(barrier_sem, 2)
            rs_cur[q] = [rs_rdma(d, 0, q) for d in (0, 1)]
            for r in rs_cur[q]:
                r.start()

        pending = vo_load(lambda d: cin(d, 0), 0)

        for s in (0, 1):
            rs_next = {}
            for q in (0, 1):
                for cp in pending:
                    cp.wait()
                for r in rs_cur[q]:
                    r.wait()
                for d in (0, 1):
                    rows = pl.ds(q * SB, SB)
                    rv[d, s % 2, rows] = (rv[d, s % 2, rows]
                                          + vo[d].astype(jnp.bfloat16))
                pending = (vo_load(lambda d: cin(d, s), 1) if q == 0
                           else vo_load(lambda d: cin(d, s + 1), 0))
                if s == 1:
                    for d in (0, 1):
                        pl.semaphore_signal(
                            credit[(d, q)], inc=1,
                            device_id=(peer[d][1],),
                            device_id_type=pl.DeviceIdType.MESH,
                        )
                    for d in (0, 1):
                        pl.semaphore_wait(credit[(d, q)], 1)
                rs_next[q] = [rs_rdma(d, s + 1, q) for d in (0, 1)]
                for r in rs_next[q]:
                    r.start()
            rs_cur = rs_next

        ag_prev = {}
        for q in (0, 1):
            for cp in pending:
                cp.wait()
            for r in rs_cur[q]:
                r.wait()
            ag_prev[q] = matmul_sub(q)
            if q == 0:
                pending = vo_load(lambda d: cin(d, 2), 1)

        for h in (1, 2):
            ag_h = {}
            for q in (0, 1):
                for r in ag_prev[q]:
                    r.wait()
                ag_h[q] = ag_rdma(h, q)
                for r in ag_h[q]:
                    r.start()
            ag_prev = ag_h
        for q in (0, 1):
            for r in ag_prev[q]:
                r.wait()

    return pl.pallas_call(
        body,
        out_shape=jax.ShapeDtypeStruct((M_PER, N_OUT), jnp.bfloat16),
        in_specs=[
            pl.BlockSpec(memory_space=pl.ANY),
            pl.BlockSpec(memory_space=pltpu.VMEM),
        ],
        out_specs=pl.BlockSpec(memory_space=pl.ANY),
        scratch_shapes=[
            pltpu.VMEM((2, 2, HC, K), jnp.bfloat16),
            pltpu.VMEM((2, SB, K), jnp.float32),
            pltpu.VMEM((2, RB, N_OUT), jnp.bfloat16),
            pltpu.SemaphoreType.DMA((2, N_DEV - 1, 2)),
            pltpu.SemaphoreType.DMA((2, N_DEV - 1, 2)),
            pltpu.SemaphoreType.DMA((2, N_DEV - 1, 2)),
            pltpu.SemaphoreType.DMA((2, N_DEV - 1, 2)),
            pltpu.SemaphoreType.DMA((2,)),
            pltpu.SemaphoreType.DMA((2,)),
            pltpu.SemaphoreType.REGULAR,
            pltpu.SemaphoreType.REGULAR,
            pltpu.SemaphoreType.REGULAR,
            pltpu.SemaphoreType.REGULAR,
        ],
        compiler_params=pltpu.CompilerParams(
            collective_id=0,
            vmem_limit_bytes=48 * 1024 * 1024,
        ),
    )(t, wb)


# device time: 302184 ns/iter; 2.2383x vs baseline; 1.0434x over previous
import jax
import jax.numpy as jnp
from jax import lax
from jax.experimental import pallas as pl
from jax.experimental.pallas import tpu as pltpu

N_DEV = 4
M_PER = 8192
MC = M_PER // N_DEV
HC = MC // 2
SB = HC // 2
K = 2048
N_OUT = 2048
RB = 256


def kernel(t, W):
    def body(t_hbm, w_hbm, out_hbm, rv, vo, obuf, wtmp, wbf,
             rs_send_sems, rs_recv_sems, ag_send_sems, ag_recv_sems,
             own_sems, out_sems, w_sem, credit_p0, credit_p1, credit_m0,
             credit_m1):
        my = lax.axis_index("i")
        right = jnp.mod(my + 1, N_DEV)
        left = jnp.mod(my + N_DEV - 1, N_DEV)
        credit = {(0, 0): credit_p0, (0, 1): credit_p1,
                  (1, 0): credit_m0, (1, 1): credit_m1}
        peer = {0: (right, left), 1: (left, right)}

        barrier_sem = pltpu.get_barrier_semaphore()
        for nbr in (left, right):
            pl.semaphore_signal(
                barrier_sem, inc=1,
                device_id=(nbr,), device_id_type=pl.DeviceIdType.MESH,
            )

        def cin(d, s):
            return (jnp.mod(my + N_DEV - s - 1, N_DEV) if d == 0
                    else jnp.mod(my + s + 1, N_DEV))

        WB = 512

        def w_block(i):
            cp = pltpu.make_async_copy(
                w_hbm.at[pl.ds(i * WB, WB)], wtmp, w_sem)
            cp.start()
            cp.wait()
            wbf[pl.ds(i * WB, WB)] = wtmp[...].astype(jnp.bfloat16)

        def vo_load(c_of_d, q):
            cps = []
            for d in (0, 1):
                cp = pltpu.make_async_copy(
                    t_hbm.at[pl.ds(c_of_d(d) * MC + d * HC + q * SB, SB)],
                    vo.at[d], own_sems.at[d])
                cp.start()
                cps.append(cp)
            return cps

        def rs_rdma(d, s, q):
            if s == 0:
                src = out_hbm.at[pl.ds(my * MC + d * HC + q * SB, SB)]
            else:
                src = rv.at[d, (s - 1) % 2, pl.ds(q * SB, SB)]
            return pltpu.make_async_remote_copy(
                src_ref=src,
                dst_ref=rv.at[d, s % 2, pl.ds(q * SB, SB)],
                send_sem=rs_send_sems.at[d, s, q],
                recv_sem=rs_recv_sems.at[d, s, q],
                device_id=(peer[d][0],),
                device_id_type=pl.DeviceIdType.MESH,
            )

        own_c = (jnp.mod(my + 1, N_DEV), jnp.mod(my + N_DEV - 1, N_DEV))

        def ag_rdma_dir(d, h, q):
            cs = (jnp.mod(my + 1 - h + N_DEV, N_DEV) if d == 0
                  else jnp.mod(my - 1 + h + N_DEV, N_DEV))
            rows = out_hbm.at[pl.ds(cs * MC + d * HC + q * SB, SB)]
            return pltpu.make_async_remote_copy(
                src_ref=rows, dst_ref=rows,
                send_sem=ag_send_sems.at[d, h, q],
                recv_sem=ag_recv_sems.at[d, h, q],
                device_id=(peer[d][0],),
                device_id_type=pl.DeviceIdType.MESH,
            )

        def ag_rdma(h, q):
            return [ag_rdma_dir(d, h, q) for d in (0, 1)]

        def matmul_sub(q):
            ags = []
            for d in (0, 1):
                for b in range(SB // RB):
                    rows = pl.ds(q * SB + b * RB, RB)
                    vrows = pl.ds(b * RB, RB)
                    blk = (rv[d, 0, rows]
                           + vo[d, vrows].astype(jnp.bfloat16))
                    acc = jnp.dot(blk, wbf[...],
                                  preferred_element_type=jnp.float32)
                    obuf[b] = acc.astype(jnp.bfloat16)
                    pltpu.make_async_copy(
                        obuf.at[b],
                        out_hbm.at[pl.ds(
                            own_c[d] * MC + d * HC + q * SB + b * RB, RB)],
                        out_sems.at[b]).start()
                for b in range(SB // RB):
                    pltpu.make_async_copy(
                        obuf.at[b], out_hbm.at[pl.ds(0, RB)],
                        out_sems.at[b]).wait()
                ag = ag_rdma_dir(d, 0, q)
                ag.start()
                ags.append(ag)
            return ags

        rs_cur = {}
        for q in (0, 1):
            cps = vo_load(lambda d: my, q)
            for cp in cps:
                cp.wait()
            j = 0
            for d in (0, 1):
                for b in range(SB // RB):
                    oslot = j % 2
                    if j >= 2:
                        pltpu.make_async_copy(
                            obuf.at[oslot], out_hbm.at[pl.ds(0, RB)],
                            out_sems.at[oslot]).wait()
                    obuf[oslot] = vo[d, pl.ds(b * RB, RB)].astype(
                        jnp.bfloat16)
                    pltpu.make_async_copy(
                        obuf.at[oslot],
                        out_hbm.at[pl.ds(
                            my * MC + d * HC + q * SB + b * RB, RB)],
                        out_sems.at[oslot]).start()
                    j += 1
            for oslot in range(2):
                pltpu.make_async_copy(
                    obuf.at[oslot], out_hbm.at[pl.ds(0, RB)],
                    out_sems.at[oslot]).wait()
            if q == 0:
                pl.semaphore_wait(barrier_sem, 2)
            rs_cur[q] = [rs_rdma(d, 0, q) for d in (0, 1)]
            for r in rs_cur[q]:
                r.start()

        pending = vo_load(lambda d: cin(d, 0), 0)
        w_block(0)

        for s in (0, 1):
            rs_next = {}
            for q in (0, 1):
                for cp in pending:
                    cp.wait()
                for r in rs_cur[q]:
                    r.wait()
                for d in (0, 1):
                    rows = pl.ds(q * SB, SB)
                    rv[d, s % 2, rows] = (rv[d, s % 2, rows]
                                          + vo[d].astype(jnp.bfloat16))
                pending = (vo_load(lambda d: cin(d, s), 1) if q == 0
                           else vo_load(lambda d: cin(d, s + 1), 0))
                if s == 1:
                    for d in (0, 1):
                        pl.semaphore_signal(
                            credit[(d, q)], inc=1,
                            device_id=(peer[d][1],),
                            device_id_type=pl.DeviceIdType.MESH,
                        )
                    for d in (0, 1):
                        pl.semaphore_wait(credit[(d, q)], 1)
                rs_next[q] = [rs_rdma(d, s + 1, q) for d in (0, 1)]
                for r in rs_next[q]:
                    r.start()
                if 2 * s + q < 3:
                    w_block(2 * s + q + 1)
            rs_cur = rs_next

        ag_prev = {}
        for q in (0, 1):
            for cp in pending:
                cp.wait()
            for r in rs_cur[q]:
                r.wait()
            ag_prev[q] = matmul_sub(q)
            if q == 0:
                pending = vo_load(lambda d: cin(d, 2), 1)

        for h in (1, 2):
            ag_h = {}
            for q in (0, 1):
                for r in ag_prev[q]:
                    r.wait()
                ag_h[q] = ag_rdma(h, q)
                for r in ag_h[q]:
                    r.start()
            ag_prev = ag_h
        for q in (0, 1):
            for r in ag_prev[q]:
                r.wait()

    return pl.pallas_call(
        body,
        out_shape=jax.ShapeDtypeStruct((M_PER, N_OUT), jnp.bfloat16),
        in_specs=[
            pl.BlockSpec(memory_space=pl.ANY),
            pl.BlockSpec(memory_space=pl.ANY),
        ],
        out_specs=pl.BlockSpec(memory_space=pl.ANY),
        scratch_shapes=[
            pltpu.VMEM((2, 2, HC, K), jnp.bfloat16),
            pltpu.VMEM((2, SB, K), jnp.float32),
            pltpu.VMEM((2, RB, N_OUT), jnp.bfloat16),
            pltpu.VMEM((512, N_OUT), jnp.float32),
            pltpu.VMEM((K, N_OUT), jnp.bfloat16),
            pltpu.SemaphoreType.DMA((2, N_DEV - 1, 2)),
            pltpu.SemaphoreType.DMA((2, N_DEV - 1, 2)),
            pltpu.SemaphoreType.DMA((2, N_DEV - 1, 2)),
            pltpu.SemaphoreType.DMA((2, N_DEV - 1, 2)),
            pltpu.SemaphoreType.DMA((2,)),
            pltpu.SemaphoreType.DMA((2,)),
            pltpu.SemaphoreType.DMA,
            pltpu.SemaphoreType.REGULAR,
            pltpu.SemaphoreType.REGULAR,
            pltpu.SemaphoreType.REGULAR,
            pltpu.SemaphoreType.REGULAR,
        ],
        compiler_params=pltpu.CompilerParams(
            collective_id=0,
            vmem_limit_bytes=56 * 1024 * 1024,
        ),
    )(t, W)


# device time: 300171 ns/iter; 2.2533x vs baseline; 1.0067x over previous
import jax
import jax.numpy as jnp
from jax import lax
from jax.experimental import pallas as pl
from jax.experimental.pallas import tpu as pltpu

N_DEV = 4
M_PER = 8192
MC = M_PER // N_DEV
HC = MC // 2
NSUB = 4
SB = HC // NSUB
K = 2048
N_OUT = 2048
QS = tuple(range(NSUB))


def kernel(t, W):
    def body(t_hbm, w_hbm, out_hbm, rv, vo, obuf, wtmp, wbf,
             rs_send_sems, rs_recv_sems, ag_send_sems, ag_recv_sems,
             own_sems, out_sems, w_sem, credit_sems):
        my = lax.axis_index("i")
        right = jnp.mod(my + 1, N_DEV)
        left = jnp.mod(my + N_DEV - 1, N_DEV)
        peer = {0: (right, left), 1: (left, right)}

        barrier_sem = pltpu.get_barrier_semaphore()
        for nbr in (left, right):
            pl.semaphore_signal(
                barrier_sem, inc=1,
                device_id=(nbr,), device_id_type=pl.DeviceIdType.MESH,
            )

        def cin(d, s):
            return (jnp.mod(my + N_DEV - s - 1, N_DEV) if d == 0
                    else jnp.mod(my + s + 1, N_DEV))

        WB = 512

        def w_block(i):
            cp = pltpu.make_async_copy(
                w_hbm.at[pl.ds(i * WB, WB)], wtmp, w_sem)
            cp.start()
            cp.wait()
            wbf[pl.ds(i * WB, WB)] = wtmp[...].astype(jnp.bfloat16)

        def vo_load(c_of_d, q):
            cps = []
            for d in (0, 1):
                cp = pltpu.make_async_copy(
                    t_hbm.at[pl.ds(c_of_d(d) * MC + d * HC + q * SB, SB)],
                    vo.at[d], own_sems.at[d])
                cp.start()
                cps.append(cp)
            return cps

        def rs_rdma(d, s, q):
            if s == 0:
                src = out_hbm.at[pl.ds(my * MC + d * HC + q * SB, SB)]
            else:
                src = rv.at[d, (s - 1) % 2, pl.ds(q * SB, SB)]
            return pltpu.make_async_remote_copy(
                src_ref=src,
                dst_ref=rv.at[d, s % 2, pl.ds(q * SB, SB)],
                send_sem=rs_send_sems.at[d, s, q],
                recv_sem=rs_recv_sems.at[d, s, q],
                device_id=(peer[d][0],),
                device_id_type=pl.DeviceIdType.MESH,
            )

        own_c = (jnp.mod(my + 1, N_DEV), jnp.mod(my + N_DEV - 1, N_DEV))

        def ag_rdma_dir(d, h, q):
            cs = (jnp.mod(my + 1 - h + N_DEV, N_DEV) if d == 0
                  else jnp.mod(my - 1 + h + N_DEV, N_DEV))
            rows = out_hbm.at[pl.ds(cs * MC + d * HC + q * SB, SB)]
            return pltpu.make_async_remote_copy(
                src_ref=rows, dst_ref=rows,
                send_sem=ag_send_sems.at[d, h, q],
                recv_sem=ag_recv_sems.at[d, h, q],
                device_id=(peer[d][0],),
                device_id_type=pl.DeviceIdType.MESH,
            )

        def matmul_sub(q):
            ags = []
            rows_q = pl.ds(q * SB, SB)
            for d in (0, 1):
                blk = rv[d, 0, rows_q] + vo[d].astype(jnp.bfloat16)
                acc = jnp.dot(blk, wbf[...],
                              preferred_element_type=jnp.float32)
                obuf[d] = acc.astype(jnp.bfloat16)
                pltpu.make_async_copy(
                    obuf.at[d],
                    out_hbm.at[pl.ds(own_c[d] * MC + d * HC + q * SB, SB)],
                    out_sems.at[d]).start()
            for d in (0, 1):
                pltpu.make_async_copy(
                    obuf.at[d], out_hbm.at[pl.ds(0, SB)],
                    out_sems.at[d]).wait()
                ag = ag_rdma_dir(d, 0, q)
                ag.start()
                ags.append(ag)
            return ags

        rs_cur = {}
        for q in QS:
            cps = vo_load(lambda d: my, q)
            for cp in cps:
                cp.wait()
            for d in (0, 1):
                obuf[d] = vo[d].astype(jnp.bfloat16)
                pltpu.make_async_copy(
                    obuf.at[d],
                    out_hbm.at[pl.ds(my * MC + d * HC + q * SB, SB)],
                    out_sems.at[d]).start()
            for d in (0, 1):
                pltpu.make_async_copy(
                    obuf.at[d], out_hbm.at[pl.ds(0, SB)],
                    out_sems.at[d]).wait()
            if q == 0:
                pl.semaphore_wait(barrier_sem, 2)
            rs_cur[q] = [rs_rdma(d, 0, q) for d in (0, 1)]
            for r in rs_cur[q]:
                r.start()

        pending = vo_load(lambda d: cin(d, 0), 0)

        for s in (0, 1):
            rs_next = {}
            for q in QS:
                for cp in pending:
                    cp.wait()
                for r in rs_cur[q]:
                    r.wait()
                for d in (0, 1):
                    rows = pl.ds(q * SB, SB)
                    rv[d, s % 2, rows] = (rv[d, s % 2, rows]
                                          + vo[d].astype(jnp.bfloat16))
                pending = (vo_load(lambda d: cin(d, s), q + 1)
                           if q < NSUB - 1
                           else vo_load(lambda d: cin(d, s + 1), 0))
                if s == 1:
                    for d in (0, 1):
                        pl.semaphore_signal(
                            credit_sems.at[d, q], inc=1,
                            device_id=(peer[d][1],),
                            device_id_type=pl.DeviceIdType.MESH,
                        )
                    for d in (0, 1):
                        pl.semaphore_wait(credit_sems.at[d, q], 1)
                rs_next[q] = [rs_rdma(d, s + 1, q) for d in (0, 1)]
                for r in rs_next[q]:
                    r.start()
                if s == 0:
                    w_block(q)
            rs_cur = rs_next

        ag_prev = {}
        for q in QS:
            for cp in pending:
                cp.wait()
            for r in rs_cur[q]:
                r.wait()
            ag_prev[q] = matmul_sub(q)
            if q < NSUB - 1:
                pending = vo_load(lambda d: cin(d, 2), q + 1)

        for h in (1, 2):
            ag_h = {}
            for q in QS:
                for r in ag_prev[q]:
                    r.wait()
                ag_h[q] = [ag_rdma_dir(d, h, q) for d in (0, 1)]
                for r in ag_h[q]:
                    r.start()
            ag_prev = ag_h
        for q in QS:
            for r in ag_prev[q]:
                r.wait()

    return pl.pallas_call(
        body,
        out_shape=jax.ShapeDtypeStruct((M_PER, N_OUT), jnp.bfloat16),
        in_specs=[
            pl.BlockSpec(memory_space=pl.ANY),
            pl.BlockSpec(memory_space=pl.ANY),
        ],
        out_specs=pl.BlockSpec(memory_space=pl.ANY),
        scratch_shapes=[
            pltpu.VMEM((2, 2, HC, K), jnp.bfloat16),
            pltpu.VMEM((2, SB, K), jnp.float32),
            pltpu.VMEM((2, SB, N_OUT), jnp.bfloat16),
            pltpu.VMEM((512, N_OUT), jnp.float32),
            pltpu.VMEM((K, N_OUT), jnp.bfloat16),
            pltpu.SemaphoreType.DMA((2, N_DEV - 1, NSUB)),
            pltpu.SemaphoreType.DMA((2, N_DEV - 1, NSUB)),
            pltpu.SemaphoreType.DMA((2, N_DEV - 1, NSUB)),
            pltpu.SemaphoreType.DMA((2, N_DEV - 1, NSUB)),
            pltpu.SemaphoreType.DMA((2,)),
            pltpu.SemaphoreType.DMA((2,)),
            pltpu.SemaphoreType.DMA,
            pltpu.SemaphoreType.REGULAR((2, NSUB)),
        ],
        compiler_params=pltpu.CompilerParams(
            collective_id=0,
            vmem_limit_bytes=56 * 1024 * 1024,
        ),
    )(t, W)
